# Initial kernel scaffold; baseline (speedup 1.0000x reference)
import jax
import jax.numpy as jnp
from jax.experimental import pallas as pl


def kernel(x, emb_weight):
    raise NotImplementedError("write your pallas kernel here")


# speedup vs baseline: 6.0730x; 6.0730x over previous
import jax
import jax.numpy as jnp
from jax.experimental import pallas as pl


def _tc_body(x_ref, r_ref, p_ref, m_ref, o_ref):
    xf = x_ref[...].astype(jnp.float32)
    m1 = jax.lax.dot_general(xf, r_ref[...], (((1,), (0,)), ((), ())),
                             preferred_element_type=jnp.float32)
    onehot = (m1 == p_ref[0:1, :]).astype(jnp.float32)
    o_ref[...] = jax.lax.dot_general(onehot, m_ref[...],
                                     (((1,), (0,)), ((), ())),
                                     preferred_element_type=jnp.float32)


def kernel(x, emb_weight):
    B, C = x.shape
    V, D = emb_weight.shape
    N = B * C
    CH = 16
    ROWS = N // CH
    LANES = CH * D
    K = CH * V

    x16 = x.reshape(ROWS, CH).astype(jnp.int32)

    t = jnp.arange(K, dtype=jnp.int32) // V
    c = jnp.arange(K, dtype=jnp.int32) % V
    rep = (jnp.arange(CH, dtype=jnp.int32)[:, None] == t[None, :]
           ).astype(jnp.float32)
    pat = jnp.broadcast_to(c.astype(jnp.float32)[None, :], (8, K))
    lane = jnp.arange(LANES, dtype=jnp.int32)
    mix = jnp.where(t[:, None] == lane[None, :] // D,
                    emb_weight[c[:, None], lane[None, :] % D],
                    0.0)

    BLK = 2048
    grid = ROWS // BLK

    out = pl.pallas_call(
        _tc_body,
        grid=(grid,),
        in_specs=[
            pl.BlockSpec((BLK, CH), lambda i: (i, 0)),
            pl.BlockSpec((CH, K), lambda i: (0, 0)),
            pl.BlockSpec((8, K), lambda i: (0, 0)),
            pl.BlockSpec((K, LANES), lambda i: (0, 0)),
        ],
        out_specs=pl.BlockSpec((BLK, LANES), lambda i: (i, 0)),
        out_shape=jax.ShapeDtypeStruct((ROWS, LANES), jnp.float32),
    )(x16, rep, pat, mix)

    return out.reshape(B, C, D)


# speedup vs baseline: 6.5090x; 1.0718x over previous
import functools

import jax
import jax.numpy as jnp
from jax import lax
from jax.experimental import pallas as pl
from jax.experimental.pallas import tpu as pltpu
from jax.experimental.pallas import tpu_sc as plsc


def _quad_table_body(wt_ref, o_ref):
    i = lax.broadcasted_iota(jnp.int32, o_ref.shape, 0)
    l = lax.broadcasted_iota(jnp.int32, o_ref.shape, 1)
    a = l // 8
    d0 = (i // 1000) % 10
    d1 = (i // 100) % 10
    d2 = (i // 10) % 10
    d3 = i % 10
    d = jnp.where(a == 0, d0, jnp.where(a == 1, d1, jnp.where(a == 2, d2, d3)))
    acc = jnp.zeros(o_ref.shape, jnp.float32)
    for c in range(10):
        acc = jnp.where(d == c, wt_ref[c:c + 1, :], acc)
    o_ref[...] = acc


def _build_quad_table(emb_weight):
    wt = jnp.tile(emb_weight, (1, 4))
    return pl.pallas_call(
        _quad_table_body,
        out_shape=jax.ShapeDtypeStruct((10000, 32), jnp.float32),
    )(wt)


_NW = 32
_CQ = 1024
_CI = 4 * _CQ


def _sc_lookup(x_hbm, t4_hbm, out_hbm, idx_v, qid_v, rows_v, sem):
    n_quads = out_hbm.shape[0]
    q_per_w = n_quads // _NW
    chunks = q_per_w // _CQ
    wid = lax.axis_index("s") * 2 + lax.axis_index("c")
    qbase = wid * q_per_w
    iota4 = lax.iota(jnp.int32, 16) * 4

    def chunk_body(cix, carry):
        q0 = qbase + cix * _CQ
        pltpu.sync_copy(x_hbm.at[pl.ds(q0 * 4, _CI)], idx_v)

        def pack_body(g, carry2):
            off = g * 64 + iota4
            g0 = plsc.load_gather(idx_v, [off])
            g1 = plsc.load_gather(idx_v, [off + 1])
            g2 = plsc.load_gather(idx_v, [off + 2])
            g3 = plsc.load_gather(idx_v, [off + 3])
            qid_v[pl.ds(g * 16, 16)] = ((g0 * 10 + g1) * 10 + g2) * 10 + g3
            return carry2

        lax.fori_loop(0, _CQ // 16, pack_body, 0, unroll=4)
        pltpu.async_copy(t4_hbm.at[qid_v], rows_v, sem).wait()
        pltpu.sync_copy(rows_v, out_hbm.at[pl.ds(q0, _CQ)])
        return carry

    lax.fori_loop(0, chunks, chunk_body, 0)


def kernel(x, emb_weight):
    B, C = x.shape
    N = B * C
    Q = N // 4
    t4 = _build_quad_table(emb_weight)
    x_flat = x.reshape(N).astype(jnp.int32)

    mesh = plsc.VectorSubcoreMesh(core_axis_name="c", subcore_axis_name="s")
    k = functools.partial(
        pl.kernel, mesh=mesh,
        compiler_params=pltpu.CompilerParams(
            needs_layout_passes=False, use_tc_tiling_on_sc=False),
        out_type=jax.ShapeDtypeStruct((Q, 32), jnp.float32),
        scratch_types=[
            pltpu.VMEM((_CI,), jnp.int32),
            pltpu.VMEM((_CQ,), jnp.int32),
            pltpu.VMEM((_CQ, 32), jnp.float32),
            pltpu.SemaphoreType.DMA,
        ],
    )(_sc_lookup)
    out = k(x_flat, t4)
    return out.reshape(B, C, 8)


# speedup vs baseline: 29.6112x; 4.5493x over previous
import functools

import jax
import jax.numpy as jnp
from jax import lax
from jax.experimental import pallas as pl
from jax.experimental.pallas import tpu as pltpu
from jax.experimental.pallas import tpu_sc as plsc


_NW = 32
_TPU = 16
_UI = 128 * _TPU
_UO = _UI * 8


def _sc_lookup(x_hbm, w_hbm, out_hbm, wf_v, idx_v, wbuf_v, sem0, sem1):
    n_units = out_hbm.shape[0] // _UO
    u_per_w = n_units // _NW
    wid = lax.axis_index("s") * 2 + lax.axis_index("c")
    u0 = wid * u_per_w
    sems = (sem0, sem1)

    pltpu.sync_copy(w_hbm, wf_v)

    def unit(u, b, do_wait):
        uu = u0 + u
        pltpu.sync_copy(x_hbm.at[pl.ds(uu * _UI, _UI)], idx_v.at[b])
        if do_wait:
            pltpu.make_async_copy(
                wbuf_v.at[b], out_hbm.at[pl.ds(uu * _UO, _UO)], sems[b]
            ).wait()

        def t_body(t, carry):
            def g_body(g, carry2):
                p = t * 128 + g * 16
                i8 = idx_v[b, pl.ds(p, 16)] * 8
                base = t * 1024 + g * 16
                for k in range(8):
                    wbuf_v[b, pl.ds(base + k * 128, 16)] = (
                        plsc.load_gather(wf_v, [i8 + k]))
                return carry2

            return lax.fori_loop(0, 8, g_body, carry, unroll=8)

        lax.fori_loop(0, _TPU, t_body, 0)
        pltpu.async_copy(
            wbuf_v.at[b], out_hbm.at[pl.ds(uu * _UO, _UO)], sems[b])

    def pair_body(p, carry):
        unit(2 * p, 0, do_wait=True)
        unit(2 * p + 1, 1, do_wait=True)
        return carry

    unit(0, 0, do_wait=False)
    unit(1, 1, do_wait=False)
    lax.fori_loop(1, u_per_w // 2, pair_body, 0)
    for b in range(2):
        pltpu.make_async_copy(
            wbuf_v.at[b], out_hbm.at[pl.ds(u0 * _UO, _UO)], sems[b]).wait()


def kernel(x, emb_weight):
    B, C = x.shape
    N = B * C
    xt_flat = x.T.reshape(N).astype(jnp.int32)
    w_flat = emb_weight.reshape(80)

    mesh = plsc.VectorSubcoreMesh(core_axis_name="c", subcore_axis_name="s")
    k = functools.partial(
        pl.kernel, mesh=mesh,
        compiler_params=pltpu.CompilerParams(
            needs_layout_passes=False, use_tc_tiling_on_sc=False),
        out_type=jax.ShapeDtypeStruct((N * 8,), jnp.float32),
        scratch_types=[
            pltpu.VMEM((80,), jnp.float32),
            pltpu.VMEM((2, _UI), jnp.int32),
            pltpu.VMEM((2, _UO), jnp.float32),
            pltpu.SemaphoreType.DMA,
            pltpu.SemaphoreType.DMA,
        ],
    )(_sc_lookup)
    flat = k(xt_flat, w_flat)
    return flat.reshape(C, B // 128, 8, 128).transpose(1, 3, 0, 2).reshape(
        B, C, 8)


# speedup vs baseline: 53.8759x; 1.8194x over previous
import functools

import jax
import jax.numpy as jnp
from jax import lax
from jax.experimental import pallas as pl
from jax.experimental.pallas import tpu as pltpu
from jax.experimental.pallas import tpu_sc as plsc


_NW = 32
_TPU = 16
_UI = 128 * _TPU
_UO = _UI * 8


def _sc_lookup(x_hbm, w_hbm, out_hbm,
               wf_v, idx_v, wbuf_v, osem0, osem1, isem0, isem1):
    n_units = out_hbm.shape[0] // _UO
    u_per_w = n_units // _NW
    wid = lax.axis_index("s") * 2 + lax.axis_index("c")
    u0 = wid * u_per_w
    osems = (osem0, osem1)
    isems = (isem0, isem1)

    pltpu.sync_copy(w_hbm, wf_v)

    def idx_fetch(u, b):
        return pltpu.async_copy(
            x_hbm.at[pl.ds((u0 + u) * _UI, _UI)], idx_v.at[b], isems[b])

    def unit(u, b, prefetch, reclaim, pred=None):
        uu = u0 + u
        pltpu.make_async_copy(
            x_hbm.at[pl.ds(uu * _UI, _UI)], idx_v.at[b], isems[b]).wait()
        if prefetch:
            if pred is None:
                idx_fetch(u + 1, 1 - b)
            else:
                @pl.when(pred)
                def _():
                    idx_fetch(u + 1, 1 - b)
        if reclaim:
            pltpu.make_async_copy(
                wbuf_v.at[b], out_hbm.at[pl.ds(uu * _UO, _UO)], osems[b]
            ).wait()

        def t_body(t, carry):
            def g_body(g, carry2):
                xv = idx_v[b, pl.ds(t * 128 + g * 16, 16)]
                base = t * 1024 + g * 16
                for k in range(8):
                    wbuf_v[b, pl.ds(base + k * 128, 16)] = (
                        plsc.load_gather(wf_v.at[k], [xv]))
                return carry2

            return lax.fori_loop(0, 8, g_body, carry, unroll=8)

        lax.fori_loop(0, _TPU, t_body, 0)
        pltpu.async_copy(
            wbuf_v.at[b], out_hbm.at[pl.ds(uu * _UO, _UO)], osems[b])

    def pair_body(p, carry):
        unit(2 * p, 0, prefetch=True, reclaim=True)
        unit(2 * p + 1, 1, prefetch=True, reclaim=True,
             pred=p < u_per_w // 2 - 1)
        return carry

    idx_fetch(0, 0)
    idx_fetch(1, 1)
    unit(0, 0, prefetch=False, reclaim=False)
    unit(1, 1, prefetch=True, reclaim=False)
    lax.fori_loop(1, u_per_w // 2, pair_body, 0)
    for b in range(2):
        pltpu.make_async_copy(
            wbuf_v.at[b], out_hbm.at[pl.ds(u0 * _UO, _UO)], osems[b]).wait()


def kernel(x, emb_weight):
    B, C = x.shape
    N = B * C
    xt_flat = x.T.reshape(N).astype(jnp.int32)
    wt_pad = jnp.pad(emb_weight.T, ((0, 0), (0, 6)))

    mesh = plsc.VectorSubcoreMesh(core_axis_name="c", subcore_axis_name="s")
    k = functools.partial(
        pl.kernel, mesh=mesh,
        compiler_params=pltpu.CompilerParams(
            needs_layout_passes=False, use_tc_tiling_on_sc=False),
        out_type=jax.ShapeDtypeStruct((N * 8,), jnp.float32),
        scratch_types=[
            pltpu.VMEM((8, 16), jnp.float32),
            pltpu.VMEM((2, _UI), jnp.int32),
            pltpu.VMEM((2, _UO), jnp.float32),
            pltpu.SemaphoreType.DMA,
            pltpu.SemaphoreType.DMA,
            pltpu.SemaphoreType.DMA,
            pltpu.SemaphoreType.DMA,
        ],
    )(_sc_lookup)
    flat = k(xt_flat, wt_pad)
    return flat.reshape(C, B // 128, 8, 128).transpose(1, 3, 0, 2).reshape(
        B, C, 8)


# speedup vs baseline: 142.8720x; 2.6519x over previous
import functools

import jax
import jax.numpy as jnp
from jax import lax
from jax.experimental import pallas as pl
from jax.experimental.pallas import tpu as pltpu
from jax.experimental.pallas import tpu_sc as plsc


_NW = 32
_TPU = 16
_UI = 128 * _TPU
_UO = _UI * 8


def _sc_lookup(x_hbm, w_hbm, out_hbm,
               wf_v, idx_v, wbuf_v, osem0, osem1, isem0, isem1):
    n_units = out_hbm.shape[0] // _UO
    u_per_w = n_units // _NW
    wid = lax.axis_index("s") * 2 + lax.axis_index("c")
    u0 = wid * u_per_w
    osems = (osem0, osem1)
    isems = (isem0, isem1)

    pltpu.sync_copy(w_hbm, wf_v)

    def idx_fetch(u, b):
        return pltpu.async_copy(
            x_hbm.at[pl.ds((u0 + u) * _UI, _UI)], idx_v.at[b], isems[b])

    def unit(u, b, prefetch, reclaim, pred=None):
        uu = u0 + u
        pltpu.make_async_copy(
            x_hbm.at[pl.ds(uu * _UI, _UI)], idx_v.at[b], isems[b]).wait()
        if prefetch:
            if pred is None:
                idx_fetch(u + 1, 1 - b)
            else:
                @pl.when(pred)
                def _():
                    idx_fetch(u + 1, 1 - b)
        if reclaim:
            pltpu.make_async_copy(
                wbuf_v.at[b], out_hbm.at[pl.ds(uu * _UO, _UO)], osems[b]
            ).wait()

        @plsc.parallel_loop(0, _TPU * 8, unroll=8)
        def g_body(g):
            t = g // 8
            gg = g % 8
            xv = idx_v[b, pl.ds(t * 128 + gg * 16, 16)]
            base = t * 1024 + gg * 16
            vals = [plsc.load_gather(wf_v.at[k], [xv]) for k in range(8)]
            for k in range(8):
                wbuf_v[b, pl.ds(base + k * 128, 16)] = vals[k]
        pltpu.async_copy(
            wbuf_v.at[b], out_hbm.at[pl.ds(uu * _UO, _UO)], osems[b])

    def pair_body(p, carry):
        unit(2 * p, 0, prefetch=True, reclaim=True)
        unit(2 * p + 1, 1, prefetch=True, reclaim=True,
             pred=p < u_per_w // 2 - 1)
        return carry

    idx_fetch(0, 0)
    idx_fetch(1, 1)
    unit(0, 0, prefetch=False, reclaim=False)
    unit(1, 1, prefetch=True, reclaim=False)
    lax.fori_loop(1, u_per_w // 2, pair_body, 0)
    for b in range(2):
        pltpu.make_async_copy(
            wbuf_v.at[b], out_hbm.at[pl.ds(u0 * _UO, _UO)], osems[b]).wait()


def kernel(x, emb_weight):
    B, C = x.shape
    N = B * C
    xt_flat = x.T.reshape(N).astype(jnp.int32)
    wt_pad = jnp.pad(emb_weight.T, ((0, 0), (0, 6)))

    mesh = plsc.VectorSubcoreMesh(core_axis_name="c", subcore_axis_name="s")
    k = functools.partial(
        pl.kernel, mesh=mesh,
        compiler_params=pltpu.CompilerParams(
            needs_layout_passes=False, use_tc_tiling_on_sc=False),
        out_type=jax.ShapeDtypeStruct((N * 8,), jnp.float32),
        scratch_types=[
            pltpu.VMEM((8, 16), jnp.float32),
            pltpu.VMEM((2, _UI), jnp.int32),
            pltpu.VMEM((2, _UO), jnp.float32),
            pltpu.SemaphoreType.DMA,
            pltpu.SemaphoreType.DMA,
            pltpu.SemaphoreType.DMA,
            pltpu.SemaphoreType.DMA,
        ],
    )(_sc_lookup)
    flat = k(xt_flat, wt_pad)
    return flat.reshape(C, B // 128, 8, 128).transpose(1, 3, 0, 2).reshape(
        B, C, 8)


# speedup vs baseline: 170.4866x; 1.1933x over previous
import functools

import jax
import jax.numpy as jnp
from jax import lax
from jax.experimental import pallas as pl
from jax.experimental.pallas import tpu as pltpu
from jax.experimental.pallas import tpu_sc as plsc


_NW = 32
_TPU = 16
_UI = 128 * _TPU
_UO = _UI * 8


def _sc_lookup(x_hbm, w_hbm, out_hbm,
               wf_v, idx_v, wbuf_v, osem0, osem1, isem0, isem1):
    n_units = out_hbm.shape[0] // _UO
    u_per_w = n_units // _NW
    wid = lax.axis_index("s") * 2 + lax.axis_index("c")
    u0 = wid * u_per_w
    osems = (osem0, osem1)
    isems = (isem0, isem1)

    pltpu.sync_copy(w_hbm, wf_v)

    def x_src(u):
        uu = u0 + u
        j = uu // 8
        q = uu % 8
        return x_hbm.at[j // 8, pl.ds(q * _TPU, _TPU), j % 8]

    def idx_fetch(u, b):
        return pltpu.async_copy(x_src(u), idx_v.at[b], isems[b])

    def unit(u, b, prefetch, reclaim, pred=None):
        uu = u0 + u
        pltpu.make_async_copy(x_src(u), idx_v.at[b], isems[b]).wait()
        if prefetch:
            if pred is None:
                idx_fetch(u + 1, 1 - b)
            else:
                @pl.when(pred)
                def _():
                    idx_fetch(u + 1, 1 - b)
        if reclaim:
            pltpu.make_async_copy(
                wbuf_v.at[b], out_hbm.at[pl.ds(uu * _UO, _UO)], osems[b]
            ).wait()

        @plsc.parallel_loop(0, _TPU * 8, unroll=8)
        def g_body(g):
            t = g // 8
            gg = g % 8
            xv = idx_v[b, t, pl.ds(gg * 16, 16)]
            base = t * 1024 + gg * 16
            vals = [plsc.load_gather(wf_v.at[k], [xv]) for k in range(8)]
            for k in range(8):
                wbuf_v[b, pl.ds(base + k * 128, 16)] = vals[k]
        pltpu.async_copy(
            wbuf_v.at[b], out_hbm.at[pl.ds(uu * _UO, _UO)], osems[b])

    def pair_body(p, carry):
        unit(2 * p, 0, prefetch=True, reclaim=True)
        unit(2 * p + 1, 1, prefetch=True, reclaim=True,
             pred=p < u_per_w // 2 - 1)
        return carry

    idx_fetch(0, 0)
    idx_fetch(1, 1)
    unit(0, 0, prefetch=False, reclaim=False)
    unit(1, 1, prefetch=True, reclaim=False)
    lax.fori_loop(1, u_per_w // 2, pair_body, 0)
    for b in range(2):
        pltpu.make_async_copy(
            wbuf_v.at[b], out_hbm.at[pl.ds(u0 * _UO, _UO)], osems[b]).wait()


def kernel(x, emb_weight):
    B, C = x.shape
    N = B * C
    x4 = (x.T.astype(jnp.int32).reshape(C // 8, 8, B // 128, 128)
          .transpose(0, 2, 1, 3))
    wt_pad = jnp.pad(emb_weight.T, ((0, 0), (0, 6)))

    mesh = plsc.VectorSubcoreMesh(core_axis_name="c", subcore_axis_name="s")
    k = functools.partial(
        pl.kernel, mesh=mesh,
        compiler_params=pltpu.CompilerParams(
            needs_layout_passes=False, use_tc_tiling_on_sc=False),
        out_type=jax.ShapeDtypeStruct((N * 8,), jnp.float32),
        scratch_types=[
            pltpu.VMEM((8, 16), jnp.float32),
            pltpu.VMEM((2, _TPU, 128), jnp.int32),
            pltpu.VMEM((2, _UO), jnp.float32),
            pltpu.SemaphoreType.DMA,
            pltpu.SemaphoreType.DMA,
            pltpu.SemaphoreType.DMA,
            pltpu.SemaphoreType.DMA,
        ],
    )(_sc_lookup)
    flat = k(x4, wt_pad)
    return flat.reshape(C, B // 128, 8, 128).transpose(1, 3, 0, 2).reshape(
        B, C, 8)


# speedup vs baseline: 170.6404x; 1.0009x over previous
import functools

import jax
import jax.numpy as jnp
from jax import lax
from jax.experimental import pallas as pl
from jax.experimental.pallas import tpu as pltpu
from jax.experimental.pallas import tpu_sc as plsc


_NW = 32
_TPU = 16
_UI = 128 * _TPU
_UO = _UI * 8


def _sc_lookup(x_hbm, w_hbm, out_hbm,
               wf_v, idx_v, wbuf_v, osem0, osem1, isem0, isem1):
    n_units = out_hbm.shape[0] // _UO
    u_per_w = n_units // _NW
    wid = lax.axis_index("s") * 2 + lax.axis_index("c")
    u0 = wid * u_per_w
    osems = (osem0, osem1)
    isems = (isem0, isem1)

    pltpu.sync_copy(w_hbm, wf_v)

    def x_src(u):
        uu = u0 + u
        j = uu // 8
        q = uu % 8
        return x_hbm.at[j // 8, pl.ds(q * _TPU, _TPU), j % 8]

    def idx_fetch(u, b):
        return pltpu.async_copy(x_src(u), idx_v.at[b], isems[b])

    def unit(u, b, prefetch, reclaim, pred=None):
        uu = u0 + u
        pltpu.make_async_copy(x_src(u), idx_v.at[b], isems[b]).wait()
        if prefetch:
            if pred is None:
                idx_fetch(u + 1, 1 - b)
            else:
                @pl.when(pred)
                def _():
                    idx_fetch(u + 1, 1 - b)
        if reclaim:
            pltpu.make_async_copy(
                wbuf_v.at[b], out_hbm.at[pl.ds(uu * _UO, _UO)], osems[b]
            ).wait()

        @plsc.parallel_loop(0, _TPU * 8, unroll=8)
        def g_body(g):
            t = g // 8
            gg = g % 8
            xv = idx_v[b, t, pl.ds(gg * 16, 16)]
            base = t * 1024 + gg * 16
            vals = [plsc.load_gather(wf_v.at[k], [xv]) for k in range(8)]
            for k in range(8):
                wbuf_v[b, pl.ds(base + k * 128, 16)] = vals[k]
        pltpu.async_copy(
            wbuf_v.at[b], out_hbm.at[pl.ds(uu * _UO, _UO)], osems[b])

    def pair_body(p, carry):
        unit(2 * p, 0, prefetch=True, reclaim=True)
        unit(2 * p + 1, 1, prefetch=True, reclaim=True,
             pred=p < u_per_w // 2 - 1)
        return carry

    idx_fetch(0, 0)
    idx_fetch(1, 1)
    unit(0, 0, prefetch=False, reclaim=False)
    unit(1, 1, prefetch=True, reclaim=False)
    lax.fori_loop(1, u_per_w // 2, pair_body, 0)
    for b in range(2):
        pltpu.make_async_copy(
            wbuf_v.at[b], out_hbm.at[pl.ds(u0 * _UO, _UO)], osems[b]).wait()


def kernel(x, emb_weight):
    B, C = x.shape
    N = B * C
    x4 = (x.T.astype(jnp.int32).reshape(C // 8, 8, B // 128, 128)
          .transpose(0, 2, 1, 3))
    wt_pad = jnp.pad(emb_weight.T, ((0, 0), (0, 6)))

    mesh = plsc.VectorSubcoreMesh(core_axis_name="c", subcore_axis_name="s")
    k = functools.partial(
        pl.kernel, mesh=mesh,
        compiler_params=pltpu.CompilerParams(
            needs_layout_passes=False, use_tc_tiling_on_sc=False,
            skip_device_barrier=True),
        out_type=jax.ShapeDtypeStruct((N * 8,), jnp.float32),
        scratch_types=[
            pltpu.VMEM((8, 16), jnp.float32),
            pltpu.VMEM((2, _TPU, 128), jnp.int32),
            pltpu.VMEM((2, _UO), jnp.float32),
            pltpu.SemaphoreType.DMA,
            pltpu.SemaphoreType.DMA,
            pltpu.SemaphoreType.DMA,
            pltpu.SemaphoreType.DMA,
        ],
    )(_sc_lookup)
    flat = k(x4, wt_pad)
    return flat.reshape(C, B // 128, 8, 128).transpose(1, 3, 0, 2).reshape(
        B, C, 8)
